# baseline (speedup 1.0000x reference)
import functools

import jax
import jax.numpy as jnp
from jax import lax
from jax.experimental import pallas as pl
from jax.experimental.pallas import tpu as pltpu
from jax.experimental.pallas import tpu_sc as plsc

N_ROWS = 32768
ROW = 2048
LANES = 16
NSLICE = ROW // LANES

BLOCK_ROWS = 1024


def _softmax_block(x_ref, o_ref):
    x = x_ref[...]
    m = jnp.max(x, axis=-1, keepdims=True)
    e = jnp.exp(x - m)
    s = jnp.sum(e, axis=-1, keepdims=True)
    o_ref[...] = e * (1.0 / s)


def _tc_softmax(x):
    return pl.pallas_call(
        _softmax_block,
        grid=(N_ROWS // BLOCK_ROWS,),
        in_specs=[pl.BlockSpec((BLOCK_ROWS, ROW), lambda i: (i, 0))],
        out_specs=pl.BlockSpec((BLOCK_ROWS, ROW), lambda i: (i, 0)),
        out_shape=jax.ShapeDtypeStruct((N_ROWS, ROW), x.dtype),
        compiler_params=pltpu.CompilerParams(
            dimension_semantics=("parallel",),
        ),
    )(x)


NW = 32
ROWS_PER_W = N_ROWS // NW
CH = 16
NCH = ROWS_PER_W // CH

_sc_mesh = plsc.VectorSubcoreMesh(core_axis_name="c", subcore_axis_name="s")


@functools.partial(
    pl.kernel,
    mesh=_sc_mesh,
    out_type=jax.ShapeDtypeStruct((N_ROWS, ROW), jnp.float32),
    scratch_types=[
        pltpu.VMEM((CH, ROW), jnp.float32),
    ],
    compiler_params=pltpu.CompilerParams(needs_layout_passes=False),
)
def _sc_softmax(x_hbm, out_hbm, buf):
    wid = lax.axis_index("s") * 2 + lax.axis_index("c")
    base = wid * ROWS_PER_W

    def chunk_body(c, carry):
        row0 = base + c * CH
        pltpu.sync_copy(x_hbm.at[pl.ds(row0, CH)], buf)

        def row_body(r, carry2):
            def expsum_body(j, s):
                v = jnp.exp(buf[r, pl.ds(j * LANES, LANES)])
                buf[r, pl.ds(j * LANES, LANES)] = v
                return s + v

            s = lax.fori_loop(
                0, NSLICE, expsum_body, jnp.zeros((LANES,), jnp.float32),
                unroll=8,
            )
            inv = 1.0 / jnp.broadcast_to(jnp.sum(s), (LANES,))

            def scale_body(j, t):
                buf[r, pl.ds(j * LANES, LANES)] = (
                    buf[r, pl.ds(j * LANES, LANES)] * inv
                )
                return t

            lax.fori_loop(0, NSLICE, scale_body, 0, unroll=8)
            return carry2

        lax.fori_loop(0, CH, row_body, 0)
        pltpu.sync_copy(buf, out_hbm.at[pl.ds(row0, CH)])
        return carry

    lax.fori_loop(0, NCH, chunk_body, 0)


def kernel(attn_weights, input_ids):
    del input_ids
    shape = attn_weights.shape
    x = attn_weights.reshape(N_ROWS, ROW)
    return _sc_softmax(x).reshape(shape)


# speedup vs baseline: 1.7271x; 1.7271x over previous
import functools

import jax
import jax.numpy as jnp
from jax import lax
from jax.experimental import pallas as pl
from jax.experimental.pallas import tpu as pltpu
from jax.experimental.pallas import tpu_sc as plsc

N_ROWS = 32768
ROW = 2048
LANES = 16
NSLICE = ROW // LANES

BLOCK_ROWS = 1024


def _softmax_block(x_ref, o_ref):
    x = x_ref[...]
    m = jnp.max(x, axis=-1, keepdims=True)
    e = jnp.exp(x - m)
    s = jnp.sum(e, axis=-1, keepdims=True)
    o_ref[...] = e * (1.0 / s)


def _tc_softmax(x):
    return pl.pallas_call(
        _softmax_block,
        grid=(N_ROWS // BLOCK_ROWS,),
        in_specs=[pl.BlockSpec((BLOCK_ROWS, ROW), lambda i: (i, 0))],
        out_specs=pl.BlockSpec((BLOCK_ROWS, ROW), lambda i: (i, 0)),
        out_shape=jax.ShapeDtypeStruct((N_ROWS, ROW), x.dtype),
        compiler_params=pltpu.CompilerParams(
            dimension_semantics=("parallel",),
        ),
    )(x)


NW = 32
ROWS_PER_W = N_ROWS // NW
CH = 8
NCH = ROWS_PER_W // CH
NBUF = 4

_sc_mesh = plsc.VectorSubcoreMesh(core_axis_name="c", subcore_axis_name="s")


def _sc_compute_chunk(buf):
    def row_body(r, carry):
        def expsum_body(j, ss):
            s0, s1, s2, s3 = ss
            o = j * 4 * LANES
            v0 = jnp.exp(buf[r, pl.ds(o, LANES)])
            v1 = jnp.exp(buf[r, pl.ds(o + LANES, LANES)])
            v2 = jnp.exp(buf[r, pl.ds(o + 2 * LANES, LANES)])
            v3 = jnp.exp(buf[r, pl.ds(o + 3 * LANES, LANES)])
            buf[r, pl.ds(o, LANES)] = v0
            buf[r, pl.ds(o + LANES, LANES)] = v1
            buf[r, pl.ds(o + 2 * LANES, LANES)] = v2
            buf[r, pl.ds(o + 3 * LANES, LANES)] = v3
            return (s0 + v0, s1 + v1, s2 + v2, s3 + v3)

        z = jnp.zeros((LANES,), jnp.float32)
        s0, s1, s2, s3 = lax.fori_loop(
            0, NSLICE // 4, expsum_body, (z, z, z, z), unroll=2
        )
        total = jnp.sum((s0 + s1) + (s2 + s3))
        inv = 1.0 / jnp.broadcast_to(total, (LANES,))

        def scale_body(j, t):
            o = j * LANES
            buf[r, pl.ds(o, LANES)] = buf[r, pl.ds(o, LANES)] * inv
            return t

        lax.fori_loop(0, NSLICE, scale_body, 0, unroll=8)
        return carry

    lax.fori_loop(0, CH, row_body, 0)


@functools.partial(
    pl.kernel,
    mesh=_sc_mesh,
    out_type=jax.ShapeDtypeStruct((N_ROWS, ROW), jnp.float32),
    scratch_types=[
        [pltpu.VMEM((CH, ROW), jnp.float32) for _ in range(NBUF)],
        [pltpu.SemaphoreType.DMA for _ in range(NBUF)],
        [pltpu.SemaphoreType.DMA for _ in range(NBUF)],
    ],
    compiler_params=pltpu.CompilerParams(needs_layout_passes=False),
)
def _sc_softmax(x_hbm, out_hbm, bufs, sins, souts):
    wid = lax.axis_index("s") * 2 + lax.axis_index("c")
    base = wid * ROWS_PER_W

    def start_in(c, b):
        pltpu.async_copy(x_hbm.at[pl.ds(base + c * CH, CH)], bufs[b], sins[b])

    def wait_in(b):
        pltpu.make_async_copy(
            x_hbm.at[pl.ds(base, CH)], bufs[b], sins[b]
        ).wait()

    def start_out(c, b):
        pltpu.async_copy(bufs[b], out_hbm.at[pl.ds(base + c * CH, CH)],
                         souts[b])

    def wait_out(b):
        pltpu.make_async_copy(
            bufs[b], out_hbm.at[pl.ds(base, CH)], souts[b]
        ).wait()

    start_in(0, 0)
    start_in(1, 1)

    def group_body(g, carry):
        for b in range(NBUF):
            s = g * NBUF + b
            wait_in(b)
            _sc_compute_chunk(bufs[b])
            start_out(s, b)
            pb = (b + 2) % NBUF

            @pl.when(s >= 2)
            def _():
                wait_out(pb)

            @pl.when(s + 2 < NCH)
            def _():
                start_in(s + 2, pb)

        return carry

    lax.fori_loop(0, NCH // NBUF, group_body, 0)
    wait_out((NCH - 2) % NBUF)
    wait_out((NCH - 1) % NBUF)


def kernel(attn_weights, input_ids):
    del input_ids
    shape = attn_weights.shape
    x = attn_weights.reshape(N_ROWS, ROW)
    return _sc_softmax(x).reshape(shape)


# speedup vs baseline: 3.0761x; 1.7811x over previous
import functools

import jax
import jax.numpy as jnp
from jax import lax
from jax.experimental import pallas as pl
from jax.experimental.pallas import tpu as pltpu
from jax.experimental.pallas import tpu_sc as plsc

N_ROWS = 32768
ROW = 2048
LANES = 16
NSLICE = ROW // LANES

BLOCK_ROWS = 1024


def _softmax_block(x_ref, o_ref):
    x = x_ref[...]
    m = jnp.max(x, axis=-1, keepdims=True)
    e = jnp.exp(x - m)
    s = jnp.sum(e, axis=-1, keepdims=True)
    o_ref[...] = e * (1.0 / s)


def _tc_softmax(x):
    return pl.pallas_call(
        _softmax_block,
        grid=(N_ROWS // BLOCK_ROWS,),
        in_specs=[pl.BlockSpec((BLOCK_ROWS, ROW), lambda i: (i, 0))],
        out_specs=pl.BlockSpec((BLOCK_ROWS, ROW), lambda i: (i, 0)),
        out_shape=jax.ShapeDtypeStruct((N_ROWS, ROW), x.dtype),
        compiler_params=pltpu.CompilerParams(
            dimension_semantics=("parallel",),
        ),
    )(x)


NW = 32
ROWS_PER_W = N_ROWS // NW
CH = 8
NCH = ROWS_PER_W // CH
NBUF = 4

_sc_mesh = plsc.VectorSubcoreMesh(core_axis_name="c", subcore_axis_name="s")


def _sc_compute_chunk(buf):
    def row_body(r, carry):
        def expsum_body(j, ss):
            s0, s1, s2, s3 = ss
            o = j * 4 * LANES
            v0 = jnp.exp(buf[r, pl.ds(o, LANES)])
            v1 = jnp.exp(buf[r, pl.ds(o + LANES, LANES)])
            v2 = jnp.exp(buf[r, pl.ds(o + 2 * LANES, LANES)])
            v3 = jnp.exp(buf[r, pl.ds(o + 3 * LANES, LANES)])
            buf[r, pl.ds(o, LANES)] = v0
            buf[r, pl.ds(o + LANES, LANES)] = v1
            buf[r, pl.ds(o + 2 * LANES, LANES)] = v2
            buf[r, pl.ds(o + 3 * LANES, LANES)] = v3
            return (s0 + v0, s1 + v1, s2 + v2, s3 + v3)

        z = jnp.zeros((LANES,), jnp.float32)
        s0, s1, s2, s3 = lax.fori_loop(
            0, NSLICE // 4, expsum_body, (z, z, z, z), unroll=2
        )
        total = jnp.sum((s0 + s1) + (s2 + s3))
        inv = 1.0 / jnp.broadcast_to(total, (LANES,))

        def scale_body(j, t):
            o = j * LANES
            buf[r, pl.ds(o, LANES)] = buf[r, pl.ds(o, LANES)] * inv
            return t

        lax.fori_loop(0, NSLICE, scale_body, 0, unroll=8)
        return carry

    lax.fori_loop(0, CH, row_body, 0)


@functools.partial(
    pl.kernel,
    mesh=_sc_mesh,
    out_type=jax.ShapeDtypeStruct((N_ROWS, ROW), jnp.float32),
    scratch_types=[
        [pltpu.VMEM((CH, ROW), jnp.float32) for _ in range(NBUF)],
        [pltpu.SemaphoreType.DMA for _ in range(NBUF)],
        [pltpu.SemaphoreType.DMA for _ in range(NBUF)],
    ],
    compiler_params=pltpu.CompilerParams(needs_layout_passes=False),
)
def _sc_softmax(x_hbm, out_hbm, bufs, sins, souts):
    wid = lax.axis_index("s") * 2 + lax.axis_index("c")
    base = wid * ROWS_PER_W

    def start_in(c, b):
        pltpu.async_copy(x_hbm.at[pl.ds(base + c * CH, CH)], bufs[b], sins[b])

    def wait_in(b):
        pltpu.make_async_copy(
            x_hbm.at[pl.ds(base, CH)], bufs[b], sins[b]
        ).wait()

    def start_out(c, b):
        pltpu.async_copy(bufs[b], out_hbm.at[pl.ds(base + c * CH, CH)],
                         souts[b])

    def wait_out(b):
        pltpu.make_async_copy(
            bufs[b], out_hbm.at[pl.ds(base, CH)], souts[b]
        ).wait()

    start_in(0, 0)
    start_in(1, 1)

    def group_body(g, carry):
        for b in range(NBUF):
            s = g * NBUF + b
            wait_in(b)
            _sc_compute_chunk(bufs[b])
            start_out(s, b)
            pb = (b + 2) % NBUF

            @pl.when(s >= 2)
            def _():
                wait_out(pb)

            @pl.when(s + 2 < NCH)
            def _():
                start_in(s + 2, pb)

        return carry

    lax.fori_loop(0, NCH // NBUF, group_body, 0)
    wait_out((NCH - 2) % NBUF)
    wait_out((NCH - 1) % NBUF)


TC_CH = 1024
TC_NSTEP = N_ROWS // TC_CH
TC_NBUF = 4


def _tc_manual_body(x_hbm, o_hbm, bufs, sins, souts):
    def start_in(c, b):
        pltpu.make_async_copy(
            x_hbm.at[pl.ds(c * TC_CH, TC_CH)], bufs.at[b], sins.at[b]
        ).start()

    def wait_in(b):
        pltpu.make_async_copy(
            x_hbm.at[pl.ds(0, TC_CH)], bufs.at[b], sins.at[b]
        ).wait()

    def start_out(c, b):
        pltpu.make_async_copy(
            bufs.at[b], o_hbm.at[pl.ds(c * TC_CH, TC_CH)], souts.at[b]
        ).start()

    def wait_out(b):
        pltpu.make_async_copy(
            bufs.at[b], o_hbm.at[pl.ds(0, TC_CH)], souts.at[b]
        ).wait()

    start_in(0, 0)
    start_in(1, 1)

    def group_body(g, carry):
        for b in range(TC_NBUF):
            s = g * TC_NBUF + b
            wait_in(b)
            x = bufs[b]
            m = jnp.max(x, axis=-1, keepdims=True)
            e = jnp.exp(x - m)
            ssum = jnp.sum(e, axis=-1, keepdims=True)
            bufs[b] = e * (1.0 / ssum)
            start_out(s, b)
            pb = (b + 2) % TC_NBUF

            @pl.when(s >= 2)
            def _():
                wait_out(pb)

            @pl.when(s + 2 < TC_NSTEP)
            def _():
                start_in(s + 2, pb)

        return carry

    lax.fori_loop(0, TC_NSTEP // TC_NBUF, group_body, 0)
    wait_out((TC_NSTEP - 2) % TC_NBUF)
    wait_out((TC_NSTEP - 1) % TC_NBUF)


def _tc_softmax_manual(x):
    return pl.pallas_call(
        _tc_manual_body,
        in_specs=[pl.BlockSpec(memory_space=pl.ANY)],
        out_specs=pl.BlockSpec(memory_space=pl.ANY),
        out_shape=jax.ShapeDtypeStruct((N_ROWS, ROW), x.dtype),
        scratch_shapes=[
            pltpu.VMEM((TC_NBUF, TC_CH, ROW), jnp.float32),
            pltpu.SemaphoreType.DMA((TC_NBUF,)),
            pltpu.SemaphoreType.DMA((TC_NBUF,)),
        ],
    )(x)


def kernel(attn_weights, input_ids):
    del input_ids
    shape = attn_weights.shape
    x = attn_weights.reshape(N_ROWS, ROW)
    return _tc_softmax_manual(x).reshape(shape)


# speedup vs baseline: 3.1077x; 1.0103x over previous
import functools

import jax
import jax.numpy as jnp
from jax import lax
from jax.experimental import pallas as pl
from jax.experimental.pallas import tpu as pltpu
from jax.experimental.pallas import tpu_sc as plsc

N_ROWS = 32768
ROW = 2048
LANES = 16
NSLICE = ROW // LANES

BLOCK_ROWS = 1024


def _softmax_block(x_ref, o_ref):
    x = x_ref[...]
    m = jnp.max(x, axis=-1, keepdims=True)
    e = jnp.exp(x - m)
    s = jnp.sum(e, axis=-1, keepdims=True)
    o_ref[...] = e * (1.0 / s)


def _tc_softmax(x):
    return pl.pallas_call(
        _softmax_block,
        grid=(N_ROWS // BLOCK_ROWS,),
        in_specs=[pl.BlockSpec((BLOCK_ROWS, ROW), lambda i: (i, 0))],
        out_specs=pl.BlockSpec((BLOCK_ROWS, ROW), lambda i: (i, 0)),
        out_shape=jax.ShapeDtypeStruct((N_ROWS, ROW), x.dtype),
        compiler_params=pltpu.CompilerParams(
            dimension_semantics=("parallel",),
        ),
    )(x)


NW = 32
ROWS_PER_W = N_ROWS // NW
CH = 8
NCH = ROWS_PER_W // CH
NBUF = 4

_sc_mesh = plsc.VectorSubcoreMesh(core_axis_name="c", subcore_axis_name="s")


def _sc_compute_chunk(buf):
    def row_body(r, carry):
        def expsum_body(j, ss):
            s0, s1, s2, s3 = ss
            o = j * 4 * LANES
            v0 = jnp.exp(buf[r, pl.ds(o, LANES)])
            v1 = jnp.exp(buf[r, pl.ds(o + LANES, LANES)])
            v2 = jnp.exp(buf[r, pl.ds(o + 2 * LANES, LANES)])
            v3 = jnp.exp(buf[r, pl.ds(o + 3 * LANES, LANES)])
            buf[r, pl.ds(o, LANES)] = v0
            buf[r, pl.ds(o + LANES, LANES)] = v1
            buf[r, pl.ds(o + 2 * LANES, LANES)] = v2
            buf[r, pl.ds(o + 3 * LANES, LANES)] = v3
            return (s0 + v0, s1 + v1, s2 + v2, s3 + v3)

        z = jnp.zeros((LANES,), jnp.float32)
        s0, s1, s2, s3 = lax.fori_loop(
            0, NSLICE // 4, expsum_body, (z, z, z, z), unroll=2
        )
        total = jnp.sum((s0 + s1) + (s2 + s3))
        inv = 1.0 / jnp.broadcast_to(total, (LANES,))

        def scale_body(j, t):
            o = j * LANES
            buf[r, pl.ds(o, LANES)] = buf[r, pl.ds(o, LANES)] * inv
            return t

        lax.fori_loop(0, NSLICE, scale_body, 0, unroll=8)
        return carry

    lax.fori_loop(0, CH, row_body, 0)


@functools.partial(
    pl.kernel,
    mesh=_sc_mesh,
    out_type=jax.ShapeDtypeStruct((N_ROWS, ROW), jnp.float32),
    scratch_types=[
        [pltpu.VMEM((CH, ROW), jnp.float32) for _ in range(NBUF)],
        [pltpu.SemaphoreType.DMA for _ in range(NBUF)],
        [pltpu.SemaphoreType.DMA for _ in range(NBUF)],
    ],
    compiler_params=pltpu.CompilerParams(needs_layout_passes=False),
)
def _sc_softmax(x_hbm, out_hbm, bufs, sins, souts):
    wid = lax.axis_index("s") * 2 + lax.axis_index("c")
    base = wid * ROWS_PER_W

    def start_in(c, b):
        pltpu.async_copy(x_hbm.at[pl.ds(base + c * CH, CH)], bufs[b], sins[b])

    def wait_in(b):
        pltpu.make_async_copy(
            x_hbm.at[pl.ds(base, CH)], bufs[b], sins[b]
        ).wait()

    def start_out(c, b):
        pltpu.async_copy(bufs[b], out_hbm.at[pl.ds(base + c * CH, CH)],
                         souts[b])

    def wait_out(b):
        pltpu.make_async_copy(
            bufs[b], out_hbm.at[pl.ds(base, CH)], souts[b]
        ).wait()

    start_in(0, 0)
    start_in(1, 1)

    def group_body(g, carry):
        for b in range(NBUF):
            s = g * NBUF + b
            wait_in(b)
            _sc_compute_chunk(bufs[b])
            start_out(s, b)
            pb = (b + 2) % NBUF

            @pl.when(s >= 2)
            def _():
                wait_out(pb)

            @pl.when(s + 2 < NCH)
            def _():
                start_in(s + 2, pb)

        return carry

    lax.fori_loop(0, NCH // NBUF, group_body, 0)
    wait_out((NCH - 2) % NBUF)
    wait_out((NCH - 1) % NBUF)


TC_CH = 512
TC_NSTEP = N_ROWS // TC_CH
TC_NBUF = 8
TC_D = 4


def _tc_manual_body(x_hbm, o_hbm, bufs, sins, souts):
    def start_in(c, b):
        pltpu.make_async_copy(
            x_hbm.at[pl.ds(c * TC_CH, TC_CH)], bufs.at[b], sins.at[b]
        ).start()

    def wait_in(b):
        pltpu.make_async_copy(
            x_hbm.at[pl.ds(0, TC_CH)], bufs.at[b], sins.at[b]
        ).wait()

    def start_out(c, b):
        pltpu.make_async_copy(
            bufs.at[b], o_hbm.at[pl.ds(c * TC_CH, TC_CH)], souts.at[b]
        ).start()

    def wait_out(b):
        pltpu.make_async_copy(
            bufs.at[b], o_hbm.at[pl.ds(0, TC_CH)], souts.at[b]
        ).wait()

    for c in range(TC_D):
        start_in(c, c)

    def group_body(g, carry):
        for b in range(TC_NBUF):
            s = g * TC_NBUF + b
            wait_in(b)
            x = bufs[b]
            m = jnp.max(x, axis=-1, keepdims=True)
            e = jnp.exp(x - m)
            ssum = jnp.sum(e, axis=-1, keepdims=True)
            bufs[b] = e * (1.0 / ssum)
            start_out(s, b)
            pb = (b + TC_D) % TC_NBUF

            @pl.when(s >= TC_NBUF - TC_D)
            def _():
                wait_out(pb)

            @pl.when(s + TC_D < TC_NSTEP)
            def _():
                start_in(s + TC_D, pb)

        return carry

    lax.fori_loop(0, TC_NSTEP // TC_NBUF, group_body, 0)
    for k in range(TC_NBUF - TC_D):
        wait_out((TC_NSTEP - (TC_NBUF - TC_D) + k) % TC_NBUF)


def _tc_softmax_manual(x):
    return pl.pallas_call(
        _tc_manual_body,
        in_specs=[pl.BlockSpec(memory_space=pl.ANY)],
        out_specs=pl.BlockSpec(memory_space=pl.ANY),
        out_shape=jax.ShapeDtypeStruct((N_ROWS, ROW), x.dtype),
        scratch_shapes=[
            pltpu.VMEM((TC_NBUF, TC_CH, ROW), jnp.float32),
            pltpu.SemaphoreType.DMA((TC_NBUF,)),
            pltpu.SemaphoreType.DMA((TC_NBUF,)),
        ],
    )(x)


def kernel(attn_weights, input_ids):
    del input_ids
    shape = attn_weights.shape
    x = attn_weights.reshape(N_ROWS, ROW)
    return _tc_softmax_manual(x).reshape(shape)
